# Initial kernel scaffold; baseline (speedup 1.0000x reference)
import jax
import jax.numpy as jnp
from jax.experimental import pallas as pl


def kernel(X, V_Adap, Wl, bl, Wr, br, We, att, bias_gat, Wfc, bfc, Wih0, Whh0, bih0, bhh0, Wih1, Whh1, bih1, bhh1, Wout, bout):
    raise NotImplementedError("write your pallas kernel here")


# speedup vs baseline: 1063.9814x; 1063.9814x over previous
import jax
import jax.numpy as jnp
from jax.experimental import pallas as pl
from jax.experimental.pallas import tpu as pltpu

_THR = 0.9
_NP = 512
_TI = 16


def _gat_body(v_ref, x_ref, wl_ref, bl_ref, wr_ref, br_ref, we_ref,
              att_ref, bg_ref, wfc_ref, bfc_ref, out_ref, alpha_scr, xl_scr):
    A = jax.nn.sigmoid(v_ref[...])
    mask = A > _THR
    x = x_ref[0]
    xl_scr[...] = jnp.dot(x, wl_ref[...], preferred_element_type=jnp.float32) + bl_ref[...]
    xr = jnp.dot(x, wr_ref[...], preferred_element_type=jnp.float32) + br_ref[...]
    we = we_ref[...]
    att = att_ref[...]

    def body(it, _):
        i0 = it * _TI
        xli = xl_scr[pl.ds(i0, _TI), :]
        Ai = jax.nn.sigmoid(v_ref[pl.ds(i0, _TI), :])
        mi = Ai > _THR
        z = xli[:, None, :] + xr[None, :, :] + Ai[:, :, None] * we[None, :, :]
        lz = jnp.maximum(z, 0.2 * z)
        ab = jnp.sum(lz * att[None, :, :], axis=-1)
        alpha_scr[pl.ds(i0, _TI), :] = jnp.where(mi, ab, -1e30)
        return 0

    jax.lax.fori_loop(0, _NP // _TI, body, 0)

    alpha = alpha_scr[...]
    amax = jnp.max(alpha, axis=0, keepdims=True)
    ex = jnp.where(mask, jnp.exp(alpha - amax), 0.0)
    den = jnp.sum(ex, axis=0, keepdims=True)
    P = ex / jnp.where(den > 0.0, den, 1.0)
    agg = jax.lax.dot_general(P, xl_scr[...], (((0,), (0,)), ((), ())),
                              preferred_element_type=jnp.float32)
    agg = agg + bg_ref[...]
    out_ref[0] = jnp.dot(agg, wfc_ref[...], preferred_element_type=jnp.float32) + bfc_ref[...]


def _proj_body(s_ref, w_ref, b_ref, o_ref):
    @pl.when(pl.program_id(0) == 0)
    def _():
        o_ref[...] = jnp.broadcast_to(b_ref[...], o_ref.shape)
    o_ref[...] += jax.lax.dot_general(
        s_ref[...], w_ref[...], (((1,), (1,)), ((), ())),
        preferred_element_type=jnp.float32)


def _lstm_body(g0_ref, whh0_ref, bhh0_ref, wih1_ref, bih1_ref,
               whh1_ref, bhh1_ref, wout_ref, bout_ref, o_ref):
    HID = whh0_ref.shape[0]

    def cell(g, c):
        i_ = jax.nn.sigmoid(g[:, 0 * HID:1 * HID])
        f_ = jax.nn.sigmoid(g[:, 1 * HID:2 * HID])
        g_ = jnp.tanh(g[:, 2 * HID:3 * HID])
        o_ = jax.nn.sigmoid(g[:, 3 * HID:4 * HID])
        c = f_ * c + i_ * g_
        return o_ * jnp.tanh(c), c

    def step(t, carry):
        h0, c0, h1, c1 = carry
        g = g0_ref[t] + jax.lax.dot_general(
            h0, whh0_ref[...], (((1,), (0,)), ((), ())),
            preferred_element_type=jnp.float32) + bhh0_ref[...]
        h0, c0 = cell(g, c0)
        g1 = (jax.lax.dot_general(h0, wih1_ref[...], (((1,), (0,)), ((), ())),
                                  preferred_element_type=jnp.float32)
              + bih1_ref[...]
              + jax.lax.dot_general(h1, whh1_ref[...], (((1,), (0,)), ((), ())),
                                    preferred_element_type=jnp.float32)
              + bhh1_ref[...])
        h1, c1 = cell(g1, c1)
        return h0, c0, h1, c1

    z = jnp.zeros((2, HID), jnp.float32)
    T = g0_ref.shape[0]
    h0, c0, h1, c1 = jax.lax.fori_loop(0, T, step, (z, z, z, z))
    o_ref[...] = jnp.dot(h1, wout_ref[...], preferred_element_type=jnp.float32) + bout_ref[...]


def kernel(X, V_Adap, Wl, bl, Wr, br, We, att, bias_gat, Wfc, bfc,
           Wih0, Whh0, bih0, bhh0, Wih1, Whh1, bih1, bhh1, Wout, bout):
    B, T, N, F = X.shape
    H = Wl.shape[1]
    OUT = Wfc.shape[1]
    HID = Whh0.shape[1]
    NC = Wout.shape[1]
    NP = _NP

    X0 = jnp.pad(X[0], ((0, 0), (0, NP - N), (0, 0)))
    Vp = jnp.pad(V_Adap, ((0, NP - N), (0, NP - N)), constant_values=-100.0)

    row = lambda v: v.reshape(1, -1)
    full = lambda shape: pl.BlockSpec(shape, lambda t: (0,) * len(shape))

    gat_out = pl.pallas_call(
        _gat_body,
        grid=(T,),
        in_specs=[
            full((NP, NP)),
            pl.BlockSpec((1, NP, F), lambda t: (t, 0, 0)),
            full((F, H)), full((1, H)), full((F, H)), full((1, H)),
            full((1, H)), full((1, H)), full((1, H)),
            full((H, OUT)), full((1, OUT)),
        ],
        out_specs=pl.BlockSpec((1, NP, OUT), lambda t: (t, 0, 0)),
        out_shape=jax.ShapeDtypeStruct((T, NP, OUT), jnp.float32),
        scratch_shapes=[pltpu.VMEM((NP, NP), jnp.float32),
                        pltpu.VMEM((NP, H), jnp.float32)],
    )(Vp, X0, Wl, row(bl), Wr, row(br), row(We), row(att), row(bias_gat),
      Wfc, row(bfc))

    S0 = gat_out[:, :N, :].reshape(T, N * OUT)
    crow = gat_out[0, NP - 1, :]
    xconst = jnp.tile(crow, N)
    S = jnp.stack([S0, jnp.broadcast_to(xconst, (T, N * OUT))], axis=1)
    S = S.reshape(2 * T, N * OUT)

    K = N * OUT
    KT = 3200
    G0 = pl.pallas_call(
        _proj_body,
        grid=(K // KT,),
        in_specs=[
            pl.BlockSpec((2 * T, KT), lambda k: (0, k)),
            pl.BlockSpec((4 * HID, KT), lambda k: (0, k)),
            pl.BlockSpec((1, 4 * HID), lambda k: (0, 0)),
        ],
        out_specs=pl.BlockSpec((2 * T, 4 * HID), lambda k: (0, 0)),
        out_shape=jax.ShapeDtypeStruct((2 * T, 4 * HID), jnp.float32),
    )(S, Wih0, row(bih0))

    Wout_p = jnp.pad(Wout, ((0, 0), (0, 128 - NC)))
    bout_p = jnp.pad(bout, ((0, 128 - NC)))

    out2 = pl.pallas_call(
        _lstm_body,
        out_shape=jax.ShapeDtypeStruct((2, 128), jnp.float32),
    )(G0.reshape(T, 2, 4 * HID), Whh0.T.reshape(HID, 4 * HID), row(bhh0),
      Wih1.T.reshape(HID, 4 * HID), row(bih1),
      Whh1.T.reshape(HID, 4 * HID), row(bhh1),
      Wout_p, row(bout_p))

    res0 = out2[0, :NC]
    resc = out2[1, :NC]
    return jnp.concatenate([res0[None, :], jnp.broadcast_to(resc, (B - 1, NC))], axis=0)


# speedup vs baseline: 1621.9079x; 1.5244x over previous
import functools

import jax
import jax.numpy as jnp
from jax import lax
from jax.experimental import pallas as pl
from jax.experimental.pallas import tpu as pltpu
from jax.experimental.pallas import tpu_sc as plsc

_THR = 0.9
_NP = 512
_NW = 32
_L = 16
_H = 128


def _prep_body(x_ref, wl_ref, bl_ref, wr_ref, brc_ref, xl_ref, xrt_ref):
    x = x_ref[0]
    xl_ref[0] = jnp.dot(x, wl_ref[...], preferred_element_type=jnp.float32) + bl_ref[...]
    xrt_ref[0] = jax.lax.dot_general(
        wr_ref[...], x, (((0,), (1,)), ((), ())),
        preferred_element_type=jnp.float32) + brc_ref[...]


def _sc_gat_body(vblk_hbm, xl_hbm, xrt_hbm, attsp_hbm, wesp_hbm, out_hbm,
                 vcol_v, xl_v, xr_v, attsp_v, wesp_v, srcs_v, avals_v,
                 alpha_v, agg_v):
    T = xl_hbm.shape[0]
    c = lax.axis_index("c")
    s = lax.axis_index("s")
    w = s * 2 + c
    pltpu.sync_copy(vblk_hbm.at[w], vcol_v)
    pltpu.sync_copy(attsp_hbm, attsp_v)
    pltpu.sync_copy(wesp_hbm, wesp_v)
    lane = lax.broadcasted_iota(jnp.int32, (_L,), 0)
    zeros = jnp.zeros((_L,), jnp.float32)

    def build(i, kcnt):
        v = vcol_v[pl.ds(i * _L, _L)]
        a = 1.0 / (1.0 + jnp.exp(-v))
        m = a > _THR
        pos = kcnt * _L + lane
        plsc.store_scatter(srcs_v, [pos],
                           jnp.full((_L,), i, jnp.int32), mask=m)
        plsc.store_scatter(avals_v, [pos], a, mask=m)
        return kcnt + m.astype(jnp.int32)

    kcnt = lax.fori_loop(0, _NP, build, jnp.zeros((_L,), jnp.int32))
    kmax = jnp.max(kcnt)

    def per_t(t, _):
        pltpu.sync_copy(xl_hbm.at[t], xl_v)
        pltpu.sync_copy(xrt_hbm.at[w, t], xr_v)

        def alpha_k(k, rmax):
            pos = k * _L + lane
            srck = plsc.load_gather(srcs_v, [pos])
            msk = k < kcnt
            avk = plsc.load_gather(avals_v, [pos])
            base = srck * _H

            def chunk(o, acc):
                g = plsc.load_gather(xl_v, [base + o], mask=msk)
                z = g + xr_v[pl.ds(o * _L, _L)] + avk * wesp_v[pl.ds(o * _L, _L)]
                return acc + attsp_v[pl.ds(o * _L, _L)] * jnp.maximum(z, 0.2 * z)

            acc = lax.fori_loop(0, _H, chunk, zeros)
            a_e = jnp.where(msk, acc, -1e30)
            plsc.store_scatter(alpha_v, [pos], a_e)
            return jnp.maximum(rmax, a_e)

        rmax = lax.fori_loop(0, kmax, alpha_k, jnp.full((_L,), -1e30, jnp.float32))

        def ex_k(k, den):
            pos = k * _L + lane
            msk = k < kcnt
            ex = jnp.where(msk, jnp.exp(plsc.load_gather(alpha_v, [pos]) - rmax), 0.0)
            plsc.store_scatter(alpha_v, [pos], ex)
            return den + ex

        den = lax.fori_loop(0, kmax, ex_k, zeros)
        inv = 1.0 / jnp.where(den > 0.0, den, 1.0)

        def zero_o(o, _v):
            agg_v[pl.ds(o * _L, _L)] = zeros
            return 0

        lax.fori_loop(0, _H, zero_o, 0)

        def agg_k(k, _v):
            pos = k * _L + lane
            srck = plsc.load_gather(srcs_v, [pos])
            msk = k < kcnt
            wk = plsc.load_gather(alpha_v, [pos]) * inv
            base = srck * _H

            def inner(o, _2):
                g = plsc.load_gather(xl_v, [base + o], mask=msk)
                sl = pl.ds(o * _L, _L)
                agg_v[sl] = agg_v[sl] + wk * g
                return 0

            lax.fori_loop(0, _H, inner, 0)
            return 0

        lax.fori_loop(0, kmax, agg_k, 0)
        pltpu.sync_copy(agg_v, out_hbm.at[w, t])
        return 0

    lax.fori_loop(0, T, per_t, 0)


def _gatout_body(aggt_ref, wfc_ref, bg_ref, bfc_ref, out_ref):
    aggt = aggt_ref[0]
    out = jax.lax.dot_general(aggt, wfc_ref[...], (((0,), (0,)), ((), ())),
                              preferred_element_type=jnp.float32)
    base = jnp.dot(bg_ref[...], wfc_ref[...],
                   preferred_element_type=jnp.float32) + bfc_ref[...]
    out_ref[0] = out + base


def _proj_body(s_ref, w_ref, b_ref, o_ref):
    @pl.when(pl.program_id(0) == 0)
    def _():
        o_ref[...] = jnp.broadcast_to(b_ref[...], o_ref.shape)
    o_ref[...] += jax.lax.dot_general(
        s_ref[...], w_ref[...], (((1,), (1,)), ((), ())),
        preferred_element_type=jnp.float32)


def _lstm_body(g0_ref, whh0_ref, bhh0_ref, wih1_ref, bih1_ref,
               whh1_ref, bhh1_ref, wout_ref, bout_ref, o_ref):
    HID = whh0_ref.shape[0]

    def cell(g, c):
        i_ = jax.nn.sigmoid(g[:, 0 * HID:1 * HID])
        f_ = jax.nn.sigmoid(g[:, 1 * HID:2 * HID])
        g_ = jnp.tanh(g[:, 2 * HID:3 * HID])
        o_ = jax.nn.sigmoid(g[:, 3 * HID:4 * HID])
        c = f_ * c + i_ * g_
        return o_ * jnp.tanh(c), c

    def step(t, carry):
        h0, c0, h1, c1 = carry
        g = g0_ref[t] + jax.lax.dot_general(
            h0, whh0_ref[...], (((1,), (0,)), ((), ())),
            preferred_element_type=jnp.float32) + bhh0_ref[...]
        h0, c0 = cell(g, c0)
        g1 = (jax.lax.dot_general(h0, wih1_ref[...], (((1,), (0,)), ((), ())),
                                  preferred_element_type=jnp.float32)
              + bih1_ref[...]
              + jax.lax.dot_general(h1, whh1_ref[...], (((1,), (0,)), ((), ())),
                                    preferred_element_type=jnp.float32)
              + bhh1_ref[...])
        h1, c1 = cell(g1, c1)
        return h0, c0, h1, c1

    z = jnp.zeros((2, HID), jnp.float32)
    T = g0_ref.shape[0]
    h0, c0, h1, c1 = lax.fori_loop(0, T, step, (z, z, z, z))
    o_ref[...] = jnp.dot(h1, wout_ref[...], preferred_element_type=jnp.float32) + bout_ref[...]


def kernel(X, V_Adap, Wl, bl, Wr, br, We, att, bias_gat, Wfc, bfc,
           Wih0, Whh0, bih0, bhh0, Wih1, Whh1, bih1, bhh1, Wout, bout):
    B, T, N, F = X.shape
    H = Wl.shape[1]
    OUT = Wfc.shape[1]
    HID = Whh0.shape[1]
    NC = Wout.shape[1]
    NP = _NP

    X0 = jnp.pad(X[0], ((0, 0), (0, NP - N), (0, 0)))
    Vp = jnp.pad(V_Adap, ((0, NP - N), (0, NP - N)), constant_values=-100.0)

    row = lambda v: v.reshape(1, -1)
    full = lambda shape: pl.BlockSpec(shape, lambda t: (0,) * len(shape))

    xl_all, xrt_all = pl.pallas_call(
        _prep_body,
        grid=(T,),
        in_specs=[
            pl.BlockSpec((1, NP, F), lambda t: (t, 0, 0)),
            full((F, H)), full((1, H)), full((F, H)), full((H, 1)),
        ],
        out_specs=[
            pl.BlockSpec((1, NP, H), lambda t: (t, 0, 0)),
            pl.BlockSpec((1, H, NP), lambda t: (t, 0, 0)),
        ],
        out_shape=[
            jax.ShapeDtypeStruct((T, NP, H), jnp.float32),
            jax.ShapeDtypeStruct((T, H, NP), jnp.float32),
        ],
    )(X0, Wl, row(bl), Wr, br.reshape(H, 1))

    vblk = Vp.reshape(NP, _NW, _L).transpose(1, 0, 2).reshape(_NW, NP * _L)
    xl_flat = xl_all.reshape(T, NP * H)
    xrt_sc = xrt_all.reshape(T, H, _NW, _L).transpose(2, 0, 1, 3).reshape(_NW, T, H * _L)
    attsp = jnp.broadcast_to(att[:, None], (H, _L)).reshape(H * _L)
    wesp = jnp.broadcast_to(We.reshape(H)[:, None], (H, _L)).reshape(H * _L)

    mesh = plsc.VectorSubcoreMesh(core_axis_name="c", subcore_axis_name="s")
    sc_gat = functools.partial(
        pl.kernel, mesh=mesh,
        compiler_params=pltpu.CompilerParams(needs_layout_passes=False),
        out_type=jax.ShapeDtypeStruct((_NW, T, H * _L), jnp.float32),
        scratch_types=[
            pltpu.VMEM((NP * _L,), jnp.float32),
            pltpu.VMEM((NP * H,), jnp.float32),
            pltpu.VMEM((H * _L,), jnp.float32),
            pltpu.VMEM((H * _L,), jnp.float32),
            pltpu.VMEM((H * _L,), jnp.float32),
            pltpu.VMEM((NP * _L,), jnp.int32),
            pltpu.VMEM((NP * _L,), jnp.float32),
            pltpu.VMEM((NP * _L,), jnp.float32),
            pltpu.VMEM((H * _L,), jnp.float32),
        ],
    )(_sc_gat_body)
    agg_sc = sc_gat(vblk, xl_flat, xrt_sc, attsp, wesp)

    aggt_all = agg_sc.reshape(_NW, T, H, _L).transpose(1, 2, 0, 3).reshape(T, H, NP)

    gat_out = pl.pallas_call(
        _gatout_body,
        grid=(T,),
        in_specs=[
            pl.BlockSpec((1, H, NP), lambda t: (t, 0, 0)),
            full((H, OUT)), full((1, H)), full((1, OUT)),
        ],
        out_specs=pl.BlockSpec((1, NP, OUT), lambda t: (t, 0, 0)),
        out_shape=jax.ShapeDtypeStruct((T, NP, OUT), jnp.float32),
    )(aggt_all, Wfc, row(bias_gat), row(bfc))

    S0 = gat_out[:, :N, :].reshape(T, N * OUT)
    crow = gat_out[0, NP - 1, :]
    xconst = jnp.tile(crow, N)
    S = jnp.stack([S0, jnp.broadcast_to(xconst, (T, N * OUT))], axis=1)
    S = S.reshape(2 * T, N * OUT)

    K = N * OUT
    KT = 3200
    G0 = pl.pallas_call(
        _proj_body,
        grid=(K // KT,),
        in_specs=[
            pl.BlockSpec((2 * T, KT), lambda k: (0, k)),
            pl.BlockSpec((4 * HID, KT), lambda k: (0, k)),
            pl.BlockSpec((1, 4 * HID), lambda k: (0, 0)),
        ],
        out_specs=pl.BlockSpec((2 * T, 4 * HID), lambda k: (0, 0)),
        out_shape=jax.ShapeDtypeStruct((2 * T, 4 * HID), jnp.float32),
    )(S, Wih0, row(bih0))

    Wout_p = jnp.pad(Wout, ((0, 0), (0, 128 - NC)))
    bout_p = jnp.pad(bout, ((0, 128 - NC)))

    out2 = pl.pallas_call(
        _lstm_body,
        out_shape=jax.ShapeDtypeStruct((2, 128), jnp.float32),
    )(G0.reshape(T, 2, 4 * HID), Whh0.T.reshape(HID, 4 * HID), row(bhh0),
      Wih1.T.reshape(HID, 4 * HID), row(bih1),
      Whh1.T.reshape(HID, 4 * HID), row(bhh1),
      Wout_p, row(bout_p))

    res0 = out2[0, :NC]
    resc = out2[1, :NC]
    return jnp.concatenate([res0[None, :], jnp.broadcast_to(resc, (B - 1, NC))], axis=0)


# speedup vs baseline: 2559.6405x; 1.5782x over previous
import functools

import jax
import jax.numpy as jnp
from jax import lax
from jax.experimental import pallas as pl
from jax.experimental.pallas import tpu as pltpu
from jax.experimental.pallas import tpu_sc as plsc

_THR = 0.9
_NP = 512
_NW = 32
_L = 16
_H = 128


def _prep_body(x_ref, wl_ref, bl_ref, wr_ref, brc_ref, xl_ref, xrt_ref):
    x = x_ref[0]
    xl_ref[0] = jnp.dot(x, wl_ref[...], preferred_element_type=jnp.float32) + bl_ref[...]
    xrt_ref[0] = jax.lax.dot_general(
        wr_ref[...], x, (((0,), (1,)), ((), ())),
        preferred_element_type=jnp.float32) + brc_ref[...]


def _sc_gat_body(vblk_hbm, xl_hbm, xrt_hbm, attsp_hbm, wesp_hbm, out_hbm,
                 vcol_v, xl_v, xr_v, attsp_v, wesp_v, srcs_v, avals_v,
                 alpha_v, agg_v):
    T = xl_hbm.shape[0]
    c = lax.axis_index("c")
    s = lax.axis_index("s")
    w = s * 2 + c
    pltpu.sync_copy(vblk_hbm.at[w], vcol_v)
    pltpu.sync_copy(attsp_hbm, attsp_v)
    pltpu.sync_copy(wesp_hbm, wesp_v)
    lane = lax.broadcasted_iota(jnp.int32, (_L,), 0)
    zeros = jnp.zeros((_L,), jnp.float32)

    def build(i, kcnt):
        v = vcol_v[pl.ds(i * _L, _L)]
        a = 1.0 / (1.0 + jnp.exp(-v))
        m = a > _THR
        pos = kcnt * _L + lane
        plsc.store_scatter(srcs_v, [pos],
                           jnp.full((_L,), i, jnp.int32), mask=m)
        plsc.store_scatter(avals_v, [pos], a, mask=m)
        return kcnt + m.astype(jnp.int32)

    kcnt = lax.fori_loop(0, _NP, build, jnp.zeros((_L,), jnp.int32))
    kmax = jnp.max(kcnt)

    def per_t(t, _):
        pltpu.sync_copy(xl_hbm.at[t], xl_v)
        pltpu.sync_copy(xrt_hbm.at[w, t], xr_v)

        def alpha_k(k, rmax):
            pos = k * _L + lane
            srck = plsc.load_gather(srcs_v, [pos])
            msk = k < kcnt
            avk = plsc.load_gather(avals_v, [pos])
            base = srck * _H

            @plsc.parallel_loop(0, _H, unroll=8, carry=zeros)
            def chunk(o, acc):
                g = plsc.load_gather(xl_v, [base + o], mask=msk)
                z = g + xr_v[pl.ds(o * _L, _L)] + avk * wesp_v[pl.ds(o * _L, _L)]
                return acc + attsp_v[pl.ds(o * _L, _L)] * jnp.maximum(z, 0.2 * z)

            acc = chunk
            a_e = jnp.where(msk, acc, -1e30)
            plsc.store_scatter(alpha_v, [pos], a_e)
            return jnp.maximum(rmax, a_e)

        rmax = lax.fori_loop(0, kmax, alpha_k, jnp.full((_L,), -1e30, jnp.float32))

        def ex_k(k, den):
            pos = k * _L + lane
            msk = k < kcnt
            ex = jnp.where(msk, jnp.exp(plsc.load_gather(alpha_v, [pos]) - rmax), 0.0)
            plsc.store_scatter(alpha_v, [pos], ex)
            return den + ex

        den = lax.fori_loop(0, kmax, ex_k, zeros)
        inv = 1.0 / jnp.where(den > 0.0, den, 1.0)

        @plsc.parallel_loop(0, _H, unroll=8)
        def _zero(o):
            agg_v[pl.ds(o * _L, _L)] = zeros

        def agg_k(k, _v):
            pos = k * _L + lane
            srck = plsc.load_gather(srcs_v, [pos])
            msk = k < kcnt
            wk = plsc.load_gather(alpha_v, [pos]) * inv
            base = srck * _H

            @plsc.parallel_loop(0, _H, unroll=8)
            def _inner(o):
                g = plsc.load_gather(xl_v, [base + o], mask=msk)
                sl = pl.ds(o * _L, _L)
                agg_v[sl] = agg_v[sl] + wk * g
            return 0

        lax.fori_loop(0, kmax, agg_k, 0)
        pltpu.sync_copy(agg_v, out_hbm.at[w, t])
        return 0

    lax.fori_loop(0, T, per_t, 0)


def _gatout_body(aggt_ref, wfc_ref, bg_ref, bfc_ref, out_ref):
    aggt = aggt_ref[0]
    out = jax.lax.dot_general(aggt, wfc_ref[...], (((0,), (0,)), ((), ())),
                              preferred_element_type=jnp.float32)
    base = jnp.dot(bg_ref[...], wfc_ref[...],
                   preferred_element_type=jnp.float32) + bfc_ref[...]
    out_ref[0] = out + base


def _proj_body(s_ref, w_ref, b_ref, o_ref):
    @pl.when(pl.program_id(0) == 0)
    def _():
        o_ref[...] = jnp.broadcast_to(b_ref[...], o_ref.shape)
    o_ref[...] += jax.lax.dot_general(
        s_ref[...], w_ref[...], (((1,), (1,)), ((), ())),
        preferred_element_type=jnp.float32)


def _lstm_body(g0_ref, whh0_ref, bhh0_ref, wih1_ref, bih1_ref,
               whh1_ref, bhh1_ref, wout_ref, bout_ref, o_ref):
    HID = whh0_ref.shape[0]

    def cell(g, c):
        i_ = jax.nn.sigmoid(g[:, 0 * HID:1 * HID])
        f_ = jax.nn.sigmoid(g[:, 1 * HID:2 * HID])
        g_ = jnp.tanh(g[:, 2 * HID:3 * HID])
        o_ = jax.nn.sigmoid(g[:, 3 * HID:4 * HID])
        c = f_ * c + i_ * g_
        return o_ * jnp.tanh(c), c

    def step(t, carry):
        h0, c0, h1, c1 = carry
        g = g0_ref[t] + jax.lax.dot_general(
            h0, whh0_ref[...], (((1,), (0,)), ((), ())),
            preferred_element_type=jnp.float32) + bhh0_ref[...]
        h0, c0 = cell(g, c0)
        g1 = (jax.lax.dot_general(h0, wih1_ref[...], (((1,), (0,)), ((), ())),
                                  preferred_element_type=jnp.float32)
              + bih1_ref[...]
              + jax.lax.dot_general(h1, whh1_ref[...], (((1,), (0,)), ((), ())),
                                    preferred_element_type=jnp.float32)
              + bhh1_ref[...])
        h1, c1 = cell(g1, c1)
        return h0, c0, h1, c1

    z = jnp.zeros((2, HID), jnp.float32)
    T = g0_ref.shape[0]
    h0, c0, h1, c1 = lax.fori_loop(0, T, step, (z, z, z, z))
    o_ref[...] = jnp.dot(h1, wout_ref[...], preferred_element_type=jnp.float32) + bout_ref[...]


def kernel(X, V_Adap, Wl, bl, Wr, br, We, att, bias_gat, Wfc, bfc,
           Wih0, Whh0, bih0, bhh0, Wih1, Whh1, bih1, bhh1, Wout, bout):
    B, T, N, F = X.shape
    H = Wl.shape[1]
    OUT = Wfc.shape[1]
    HID = Whh0.shape[1]
    NC = Wout.shape[1]
    NP = _NP

    X0 = jnp.pad(X[0], ((0, 0), (0, NP - N), (0, 0)))
    Vp = jnp.pad(V_Adap, ((0, NP - N), (0, NP - N)), constant_values=-100.0)

    row = lambda v: v.reshape(1, -1)
    full = lambda shape: pl.BlockSpec(shape, lambda t: (0,) * len(shape))

    xl_all, xrt_all = pl.pallas_call(
        _prep_body,
        grid=(T,),
        in_specs=[
            pl.BlockSpec((1, NP, F), lambda t: (t, 0, 0)),
            full((F, H)), full((1, H)), full((F, H)), full((H, 1)),
        ],
        out_specs=[
            pl.BlockSpec((1, NP, H), lambda t: (t, 0, 0)),
            pl.BlockSpec((1, H, NP), lambda t: (t, 0, 0)),
        ],
        out_shape=[
            jax.ShapeDtypeStruct((T, NP, H), jnp.float32),
            jax.ShapeDtypeStruct((T, H, NP), jnp.float32),
        ],
    )(X0, Wl, row(bl), Wr, br.reshape(H, 1))

    vblk = Vp.reshape(NP, _NW, _L).transpose(1, 0, 2).reshape(_NW, NP * _L)
    xl_flat = xl_all.reshape(T, NP * H)
    xrt_sc = xrt_all.reshape(T, H, _NW, _L).transpose(2, 0, 1, 3).reshape(_NW, T, H * _L)
    attsp = jnp.broadcast_to(att[:, None], (H, _L)).reshape(H * _L)
    wesp = jnp.broadcast_to(We.reshape(H)[:, None], (H, _L)).reshape(H * _L)

    mesh = plsc.VectorSubcoreMesh(core_axis_name="c", subcore_axis_name="s")
    sc_gat = functools.partial(
        pl.kernel, mesh=mesh,
        compiler_params=pltpu.CompilerParams(needs_layout_passes=False),
        out_type=jax.ShapeDtypeStruct((_NW, T, H * _L), jnp.float32),
        scratch_types=[
            pltpu.VMEM((NP * _L,), jnp.float32),
            pltpu.VMEM((NP * H,), jnp.float32),
            pltpu.VMEM((H * _L,), jnp.float32),
            pltpu.VMEM((H * _L,), jnp.float32),
            pltpu.VMEM((H * _L,), jnp.float32),
            pltpu.VMEM((NP * _L,), jnp.int32),
            pltpu.VMEM((NP * _L,), jnp.float32),
            pltpu.VMEM((NP * _L,), jnp.float32),
            pltpu.VMEM((H * _L,), jnp.float32),
        ],
    )(_sc_gat_body)
    agg_sc = sc_gat(vblk, xl_flat, xrt_sc, attsp, wesp)

    aggt_all = agg_sc.reshape(_NW, T, H, _L).transpose(1, 2, 0, 3).reshape(T, H, NP)

    gat_out = pl.pallas_call(
        _gatout_body,
        grid=(T,),
        in_specs=[
            pl.BlockSpec((1, H, NP), lambda t: (t, 0, 0)),
            full((H, OUT)), full((1, H)), full((1, OUT)),
        ],
        out_specs=pl.BlockSpec((1, NP, OUT), lambda t: (t, 0, 0)),
        out_shape=jax.ShapeDtypeStruct((T, NP, OUT), jnp.float32),
    )(aggt_all, Wfc, row(bias_gat), row(bfc))

    S0 = gat_out[:, :N, :].reshape(T, N * OUT)
    crow = gat_out[0, NP - 1, :]
    xconst = jnp.tile(crow, N)
    S = jnp.stack([S0, jnp.broadcast_to(xconst, (T, N * OUT))], axis=1)
    S = S.reshape(2 * T, N * OUT)

    K = N * OUT
    KT = 3200
    G0 = pl.pallas_call(
        _proj_body,
        grid=(K // KT,),
        in_specs=[
            pl.BlockSpec((2 * T, KT), lambda k: (0, k)),
            pl.BlockSpec((4 * HID, KT), lambda k: (0, k)),
            pl.BlockSpec((1, 4 * HID), lambda k: (0, 0)),
        ],
        out_specs=pl.BlockSpec((2 * T, 4 * HID), lambda k: (0, 0)),
        out_shape=jax.ShapeDtypeStruct((2 * T, 4 * HID), jnp.float32),
    )(S, Wih0, row(bih0))

    Wout_p = jnp.pad(Wout, ((0, 0), (0, 128 - NC)))
    bout_p = jnp.pad(bout, ((0, 128 - NC)))

    out2 = pl.pallas_call(
        _lstm_body,
        out_shape=jax.ShapeDtypeStruct((2, 128), jnp.float32),
    )(G0.reshape(T, 2, 4 * HID), Whh0.T.reshape(HID, 4 * HID), row(bhh0),
      Wih1.T.reshape(HID, 4 * HID), row(bih1),
      Whh1.T.reshape(HID, 4 * HID), row(bhh1),
      Wout_p, row(bout_p))

    res0 = out2[0, :NC]
    resc = out2[1, :NC]
    return jnp.concatenate([res0[None, :], jnp.broadcast_to(resc, (B - 1, NC))], axis=0)


# speedup vs baseline: 2572.9154x; 1.0052x over previous
import functools

import jax
import jax.numpy as jnp
from jax import lax
from jax.experimental import pallas as pl
from jax.experimental.pallas import tpu as pltpu
from jax.experimental.pallas import tpu_sc as plsc

_THR = 0.9
_NP = 512
_NW = 32
_L = 16
_H = 128


def _prep_body(x_ref, wl_ref, bl_ref, wr_ref, brc_ref, xl_ref, xrt_ref):
    x = x_ref[0]
    xl_ref[0] = jnp.dot(x, wl_ref[...], preferred_element_type=jnp.float32) + bl_ref[...]
    xrt_ref[0] = jax.lax.dot_general(
        wr_ref[...], x, (((0,), (1,)), ((), ())),
        preferred_element_type=jnp.float32) + brc_ref[...]


def _sc_gat_body(vblk_hbm, xl_hbm, xrt_hbm, attsp_hbm, wesp_hbm, out_hbm,
                 vcol_v, xl_v, xr_v, attsp_v, wesp_v, srcs_v, avals_v,
                 alpha_v, agg_v):
    T = xl_hbm.shape[0]
    c = lax.axis_index("c")
    s = lax.axis_index("s")
    w = s * 2 + c
    pltpu.sync_copy(vblk_hbm.at[w], vcol_v)
    pltpu.sync_copy(attsp_hbm, attsp_v)
    pltpu.sync_copy(wesp_hbm, wesp_v)
    lane = lax.broadcasted_iota(jnp.int32, (_L,), 0)
    zeros = jnp.zeros((_L,), jnp.float32)

    def build(i, kcnt):
        v = vcol_v[pl.ds(i * _L, _L)]
        a = 1.0 / (1.0 + jnp.exp(-v))
        m = a > _THR
        pos = kcnt * _L + lane
        plsc.store_scatter(srcs_v, [pos],
                           jnp.full((_L,), i, jnp.int32), mask=m)
        plsc.store_scatter(avals_v, [pos], a, mask=m)
        return kcnt + m.astype(jnp.int32)

    kcnt = lax.fori_loop(0, _NP, build, jnp.zeros((_L,), jnp.int32))
    kmax = jnp.max(kcnt)

    def per_t(t, _):
        pltpu.sync_copy(xl_hbm.at[t], xl_v)
        pltpu.sync_copy(xrt_hbm.at[w, t], xr_v)

        @plsc.parallel_loop(0, kmax, carry=jnp.full((_L,), -1e30, jnp.float32))
        def alpha_k(k, rmax):
            pos = k * _L + lane
            srck = plsc.load_gather(srcs_v, [pos])
            msk = k < kcnt
            avk = plsc.load_gather(avals_v, [pos])
            base = srck * _H

            @plsc.parallel_loop(0, _H, unroll=8, carry=zeros)
            def chunk(o, acc):
                g = plsc.load_gather(xl_v, [base + o], mask=msk)
                z = g + xr_v[pl.ds(o * _L, _L)] + avk * wesp_v[pl.ds(o * _L, _L)]
                return acc + attsp_v[pl.ds(o * _L, _L)] * jnp.maximum(z, 0.2 * z)

            a_e = jnp.where(msk, chunk, -1e30)
            plsc.store_scatter(alpha_v, [pos], a_e)
            return jnp.maximum(rmax, a_e)

        rmax = alpha_k

        @plsc.parallel_loop(0, kmax, unroll=4, carry=zeros)
        def ex_k(k, den):
            pos = k * _L + lane
            msk = k < kcnt
            ex = jnp.where(msk, jnp.exp(plsc.load_gather(alpha_v, [pos]) - rmax), 0.0)
            plsc.store_scatter(alpha_v, [pos], ex)
            return den + ex

        den = ex_k
        inv = 1.0 / jnp.where(den > 0.0, den, 1.0)

        @plsc.parallel_loop(0, _H, unroll=8)
        def _zero(o):
            agg_v[pl.ds(o * _L, _L)] = zeros

        def agg_k(k, _v):
            pos = k * _L + lane
            srck = plsc.load_gather(srcs_v, [pos])
            msk = k < kcnt
            wk = plsc.load_gather(alpha_v, [pos]) * inv
            base = srck * _H

            @plsc.parallel_loop(0, _H, unroll=8)
            def _inner(o):
                g = plsc.load_gather(xl_v, [base + o], mask=msk)
                sl = pl.ds(o * _L, _L)
                agg_v[sl] = agg_v[sl] + wk * g
            return 0

        lax.fori_loop(0, kmax, agg_k, 0)
        pltpu.sync_copy(agg_v, out_hbm.at[w, t])
        return 0

    lax.fori_loop(0, T, per_t, 0)


def _gatout_body(aggt_ref, wfc_ref, bg_ref, bfc_ref, out_ref):
    aggt = aggt_ref[0]
    out = jax.lax.dot_general(aggt, wfc_ref[...], (((0,), (0,)), ((), ())),
                              preferred_element_type=jnp.float32)
    base = jnp.dot(bg_ref[...], wfc_ref[...],
                   preferred_element_type=jnp.float32) + bfc_ref[...]
    out_ref[0] = out + base


def _proj_body(s_ref, w_ref, b_ref, o_ref):
    @pl.when(pl.program_id(0) == 0)
    def _():
        o_ref[...] = jnp.broadcast_to(b_ref[...], o_ref.shape)
    o_ref[...] += jax.lax.dot_general(
        s_ref[...], w_ref[...], (((1,), (1,)), ((), ())),
        preferred_element_type=jnp.float32)


def _lstm_body(g0_ref, whh0_ref, bhh0_ref, wih1_ref, bih1_ref,
               whh1_ref, bhh1_ref, wout_ref, bout_ref, o_ref):
    HID = whh0_ref.shape[0]

    def cell(g, c):
        i_ = jax.nn.sigmoid(g[:, 0 * HID:1 * HID])
        f_ = jax.nn.sigmoid(g[:, 1 * HID:2 * HID])
        g_ = jnp.tanh(g[:, 2 * HID:3 * HID])
        o_ = jax.nn.sigmoid(g[:, 3 * HID:4 * HID])
        c = f_ * c + i_ * g_
        return o_ * jnp.tanh(c), c

    def step(t, carry):
        h0, c0, h1, c1 = carry
        g = g0_ref[t] + jax.lax.dot_general(
            h0, whh0_ref[...], (((1,), (0,)), ((), ())),
            preferred_element_type=jnp.float32) + bhh0_ref[...]
        h0, c0 = cell(g, c0)
        g1 = (jax.lax.dot_general(h0, wih1_ref[...], (((1,), (0,)), ((), ())),
                                  preferred_element_type=jnp.float32)
              + bih1_ref[...]
              + jax.lax.dot_general(h1, whh1_ref[...], (((1,), (0,)), ((), ())),
                                    preferred_element_type=jnp.float32)
              + bhh1_ref[...])
        h1, c1 = cell(g1, c1)
        return h0, c0, h1, c1

    z = jnp.zeros((2, HID), jnp.float32)
    T = g0_ref.shape[0]
    h0, c0, h1, c1 = lax.fori_loop(0, T, step, (z, z, z, z))
    o_ref[...] = jnp.dot(h1, wout_ref[...], preferred_element_type=jnp.float32) + bout_ref[...]


def kernel(X, V_Adap, Wl, bl, Wr, br, We, att, bias_gat, Wfc, bfc,
           Wih0, Whh0, bih0, bhh0, Wih1, Whh1, bih1, bhh1, Wout, bout):
    B, T, N, F = X.shape
    H = Wl.shape[1]
    OUT = Wfc.shape[1]
    HID = Whh0.shape[1]
    NC = Wout.shape[1]
    NP = _NP

    X0 = jnp.pad(X[0], ((0, 0), (0, NP - N), (0, 0)))
    Vp = jnp.pad(V_Adap, ((0, NP - N), (0, NP - N)), constant_values=-100.0)

    row = lambda v: v.reshape(1, -1)
    full = lambda shape: pl.BlockSpec(shape, lambda t: (0,) * len(shape))

    xl_all, xrt_all = pl.pallas_call(
        _prep_body,
        grid=(T,),
        in_specs=[
            pl.BlockSpec((1, NP, F), lambda t: (t, 0, 0)),
            full((F, H)), full((1, H)), full((F, H)), full((H, 1)),
        ],
        out_specs=[
            pl.BlockSpec((1, NP, H), lambda t: (t, 0, 0)),
            pl.BlockSpec((1, H, NP), lambda t: (t, 0, 0)),
        ],
        out_shape=[
            jax.ShapeDtypeStruct((T, NP, H), jnp.float32),
            jax.ShapeDtypeStruct((T, H, NP), jnp.float32),
        ],
    )(X0, Wl, row(bl), Wr, br.reshape(H, 1))

    vblk = Vp.reshape(NP, _NW, _L).transpose(1, 0, 2).reshape(_NW, NP * _L)
    xl_flat = xl_all.reshape(T, NP * H)
    xrt_sc = xrt_all.reshape(T, H, _NW, _L).transpose(2, 0, 1, 3).reshape(_NW, T, H * _L)
    attsp = jnp.broadcast_to(att[:, None], (H, _L)).reshape(H * _L)
    wesp = jnp.broadcast_to(We.reshape(H)[:, None], (H, _L)).reshape(H * _L)

    mesh = plsc.VectorSubcoreMesh(core_axis_name="c", subcore_axis_name="s")
    sc_gat = functools.partial(
        pl.kernel, mesh=mesh,
        compiler_params=pltpu.CompilerParams(needs_layout_passes=False),
        out_type=jax.ShapeDtypeStruct((_NW, T, H * _L), jnp.float32),
        scratch_types=[
            pltpu.VMEM((NP * _L,), jnp.float32),
            pltpu.VMEM((NP * H,), jnp.float32),
            pltpu.VMEM((H * _L,), jnp.float32),
            pltpu.VMEM((H * _L,), jnp.float32),
            pltpu.VMEM((H * _L,), jnp.float32),
            pltpu.VMEM((NP * _L,), jnp.int32),
            pltpu.VMEM((NP * _L,), jnp.float32),
            pltpu.VMEM((NP * _L,), jnp.float32),
            pltpu.VMEM((H * _L,), jnp.float32),
        ],
    )(_sc_gat_body)
    agg_sc = sc_gat(vblk, xl_flat, xrt_sc, attsp, wesp)

    aggt_all = agg_sc.reshape(_NW, T, H, _L).transpose(1, 2, 0, 3).reshape(T, H, NP)

    gat_out = pl.pallas_call(
        _gatout_body,
        grid=(T,),
        in_specs=[
            pl.BlockSpec((1, H, NP), lambda t: (t, 0, 0)),
            full((H, OUT)), full((1, H)), full((1, OUT)),
        ],
        out_specs=pl.BlockSpec((1, NP, OUT), lambda t: (t, 0, 0)),
        out_shape=jax.ShapeDtypeStruct((T, NP, OUT), jnp.float32),
    )(aggt_all, Wfc, row(bias_gat), row(bfc))

    S0 = gat_out[:, :N, :].reshape(T, N * OUT)
    crow = gat_out[0, NP - 1, :]
    xconst = jnp.tile(crow, N)
    S = jnp.stack([S0, jnp.broadcast_to(xconst, (T, N * OUT))], axis=1)
    S = S.reshape(2 * T, N * OUT)

    K = N * OUT
    KT = 3200
    G0 = pl.pallas_call(
        _proj_body,
        grid=(K // KT,),
        in_specs=[
            pl.BlockSpec((2 * T, KT), lambda k: (0, k)),
            pl.BlockSpec((4 * HID, KT), lambda k: (0, k)),
            pl.BlockSpec((1, 4 * HID), lambda k: (0, 0)),
        ],
        out_specs=pl.BlockSpec((2 * T, 4 * HID), lambda k: (0, 0)),
        out_shape=jax.ShapeDtypeStruct((2 * T, 4 * HID), jnp.float32),
    )(S, Wih0, row(bih0))

    Wout_p = jnp.pad(Wout, ((0, 0), (0, 128 - NC)))
    bout_p = jnp.pad(bout, ((0, 128 - NC)))

    out2 = pl.pallas_call(
        _lstm_body,
        out_shape=jax.ShapeDtypeStruct((2, 128), jnp.float32),
    )(G0.reshape(T, 2, 4 * HID), Whh0.T.reshape(HID, 4 * HID), row(bhh0),
      Wih1.T.reshape(HID, 4 * HID), row(bih1),
      Whh1.T.reshape(HID, 4 * HID), row(bhh1),
      Wout_p, row(bout_p))

    res0 = out2[0, :NC]
    resc = out2[1, :NC]
    return jnp.concatenate([res0[None, :], jnp.broadcast_to(resc, (B - 1, NC))], axis=0)
